# baseline (speedup 1.0000x reference)
import functools

import jax
import jax.numpy as jnp
from jax import lax
from jax.experimental import pallas as pl
from jax.experimental.pallas import tpu as pltpu
from jax.experimental.pallas import tpu_sc as plsc

_BATCH = 4096
_HIST = 50
_DIM = 64
_PDIM = 128
_VOCAB = 1000000
_NW = 32

_CHUNK = 320
_ROWS_PER_W = (_BATCH * _HIST) // _NW
_NCHUNK = _ROWS_PER_W // _CHUNK
_NBUF = 3


_TBLK = 32768
_TGRID = -(-_VOCAB // _TBLK)


def _build_detile():
    def detile_body(tt_ref, s_ref):
        s_ref[:, : _DIM] = jnp.transpose(tt_ref[...])

    return pl.pallas_call(
        detile_body,
        grid=(_TGRID,),
        in_specs=[pl.BlockSpec((_DIM, _TBLK), lambda i: (0, i))],
        out_specs=pl.BlockSpec((_TBLK, _PDIM), lambda i: (i, 0)),
        out_shape=jax.ShapeDtypeStruct((_VOCAB, _PDIM), jnp.float32),
    )


def _build_gather():
    mesh = plsc.VectorSubcoreMesh(core_axis_name="c", subcore_axis_name="s")

    @functools.partial(
        pl.kernel,
        out_type=jax.ShapeDtypeStruct((_NW, _NCHUNK, _CHUNK, _DIM), jnp.float32),
        mesh=mesh,
        scratch_types=[
            pltpu.VMEM((_NCHUNK, _CHUNK), jnp.int32),
            pltpu.VMEM((_NBUF, _CHUNK, _PDIM), jnp.float32),
            pltpu.SemaphoreType.DMA((_NBUF,)),
            pltpu.SemaphoreType.DMA((_NBUF,)),
        ],
        compiler_params=pltpu.CompilerParams(use_tc_tiling_on_sc=False),
    )
    def gather_kernel(table_hbm, idx_hbm, out_hbm, idx_v, rows_v, gsem, osem):
        wid = lax.axis_index("s") * 2 + lax.axis_index("c")
        pltpu.sync_copy(idx_hbm.at[wid], idx_v)

        def fire_gather(j):
            s = j % _NBUF
            pltpu.make_async_copy(
                table_hbm.at[idx_v.at[j]], rows_v.at[s], gsem.at[s]).start()

        def wait_gather(j):
            s = j % _NBUF
            pltpu.make_async_copy(
                table_hbm.at[idx_v.at[j]], rows_v.at[s], gsem.at[s]).wait()

        def out_copy(j):
            s = j % _NBUF
            return pltpu.make_async_copy(
                rows_v.at[s, :, pl.ds(0, _DIM)], out_hbm.at[wid, j],
                osem.at[s])

        fire_gather(0)
        fire_gather(1)
        for j in range(_NCHUNK):
            wait_gather(j)
            nxt = j + 2
            if nxt < _NCHUNK:
                if nxt >= _NBUF:
                    out_copy(nxt - _NBUF).wait()
                fire_gather(nxt)
            out_copy(j).start()
        for j in range(_NCHUNK - _NBUF, _NCHUNK):
            out_copy(j).wait()

    return gather_kernel


_detile = _build_detile()
_gather = _build_gather()


def kernel(inputs, embeddings):
    tt = jnp.transpose(embeddings)
    table = _detile(tt)
    idx = inputs.astype(jnp.int32).reshape(_NW, _NCHUNK, _CHUNK)
    out = _gather(table, idx)
    return out.reshape(_BATCH, _HIST, _DIM)
